# Initial kernel scaffold; baseline (speedup 1.0000x reference)
import jax
import jax.numpy as jnp
from jax.experimental import pallas as pl


def kernel(x, edge_index, cand, W0, b0, W1, b1, W2, b2, Ws0, bs0, Ws1, bs1, Ws2, bs2, We0, be0, We1, be1, We2, be2):
    raise NotImplementedError("write your pallas kernel here")


# speedup vs baseline: 9.9679x; 9.9679x over previous
import functools

import jax
import jax.numpy as jnp
from jax import lax
from jax.experimental import pallas as pl
from jax.experimental.pallas import tpu as pltpu
from jax.experimental.pallas import tpu_sc as plsc

N = 10000
C = 1024
D = 128
NT = N + C
E = 320000
ALPHA = 0.2

NC = 2
NS = 16
LN = 16
NW = NC * NS
K = 128
NCH = -(-E // (NW * K))
EP = NW * K * NCH
PW = EP // NW
RPT = 704
NTP = NS * RPT
TRASH = NT

def _fill(buf, nrows, value):
    def body(i, _):
        for j in range(D // LN):
            buf[i, pl.ds(j * LN, LN)] = jnp.full((LN,), value, jnp.float32)
        return 0
    lax.fori_loop(0, nrows, body, 0)


def _zero_acc_slice(acc, rows, s):
    base = s * RPT
    for t in range(RPT // K):
        pltpu.sync_copy(rows, acc.at[pl.ds(base + t * K, K)])
    rem = RPT - (RPT // K) * K
    if rem:
        pltpu.sync_copy(rows.at[pl.ds(0, rem)],
                        acc.at[pl.ds(base + (RPT // K) * K, rem)])


def _edge_agg_body(g_hbm, src_hbm, dst_hbm, out_hbm,
                   acc, idx_s, idx_d, rows, sem):
    c = lax.axis_index("c")
    s = lax.axis_index("s")
    w = c * NS + s

    _fill(rows, K, 0.0)
    _zero_acc_slice(acc, rows, s)
    plsc.subcore_barrier()

    pltpu.sync_copy(src_hbm.at[w], idx_s)
    pltpu.sync_copy(dst_hbm.at[w], idx_d)

    def body(i, _):
        pltpu.async_copy(g_hbm.at[idx_s.at[i]], rows, sem).wait()
        pltpu.sync_copy(rows, acc.at[idx_d.at[i]], add=True)
        return 0
    lax.fori_loop(0, NCH, body, 0)

    plsc.subcore_barrier()
    pltpu.sync_copy(acc.at[pl.ds(s * RPT, RPT)],
                    out_hbm.at[c, pl.ds(s * RPT, RPT)])


def _deg_body(dst_hbm, out_hbm, acc, idx_d, rows):
    c = lax.axis_index("c")
    s = lax.axis_index("s")
    w = c * NS + s

    _fill(rows, K, 0.0)
    _zero_acc_slice(acc, rows, s)
    _fill(rows, K, 1.0)
    plsc.subcore_barrier()

    pltpu.sync_copy(dst_hbm.at[w], idx_d)

    def body(i, _):
        pltpu.sync_copy(rows, acc.at[idx_d.at[i]], add=True)
        return 0
    lax.fori_loop(0, NCH, body, 0)

    plsc.subcore_barrier()
    pltpu.sync_copy(acc.at[pl.ds(s * RPT, RPT)],
                    out_hbm.at[c, pl.ds(s * RPT, RPT)])


@functools.cache
def _sc_kernels():
    mesh = plsc.VectorSubcoreMesh(
        core_axis_name="c", subcore_axis_name="s",
        num_cores=NC, num_subcores=NS)
    edge_agg = pl.kernel(
        _edge_agg_body,
        out_type=jax.ShapeDtypeStruct((NC, NTP, D), jnp.float32),
        mesh=mesh,
        scratch_types=[
            pltpu.VMEM_SHARED((NTP, D), jnp.float32),
            pltpu.VMEM((NCH, K), jnp.int32),
            pltpu.VMEM((NCH, K), jnp.int32),
            pltpu.VMEM((K, D), jnp.float32),
            pltpu.SemaphoreType.DMA,
        ],
    )
    deg_hist = pl.kernel(
        _deg_body,
        out_type=jax.ShapeDtypeStruct((NC, NTP, D), jnp.float32),
        mesh=mesh,
        scratch_types=[
            pltpu.VMEM_SHARED((NTP, D), jnp.float32),
            pltpu.VMEM((NCH, K), jnp.int32),
            pltpu.VMEM((K, D), jnp.float32),
        ],
    )
    return edge_agg, deg_hist


def _leaky(x):
    return jnp.where(x >= 0, x, ALPHA * x)


def _prep_body(h_ref, w0_ref, degp_ref, g0_ref, dinv_ref):
    deg = degp_ref[0, :NT, :] + degp_ref[1, :NT, :] + 1.0
    dinv = lax.rsqrt(deg)
    h2 = jnp.dot(h_ref[...], w0_ref[...], preferred_element_type=jnp.float32)
    dinv_ref[...] = dinv
    g0_ref[...] = h2 * dinv


def _mid_body(p_ref, g_ref, dinv_ref, b_ref, wn_ref, gn_ref):
    s = p_ref[0, :NT, :] + p_ref[1, :NT, :] + g_ref[...]
    a = _leaky(dinv_ref[...] * s + b_ref[...])
    gn_ref[...] = jnp.dot(a, wn_ref[...],
                          preferred_element_type=jnp.float32) * dinv_ref[...]


def _mlp_scores(a, W1, b1, W2, b2, W3, b3):
    t = _leaky(jnp.dot(a, W1, preferred_element_type=jnp.float32) + b1)
    t = _leaky(jnp.dot(t, W2, preferred_element_type=jnp.float32) + b2)
    v = jnp.dot(t, W3, preferred_element_type=jnp.float32) + b3
    return jnp.clip(v, 0.0, 6.0)


def _softmax0(v, mask):
    mv = jnp.where(mask, v, -1e30)
    m = jnp.max(mv, axis=0, keepdims=True)
    p = jnp.exp(mv - m)
    return p / jnp.sum(p, axis=0, keepdims=True)


def _fin_body(p_ref, g_ref, dinv_ref, b2_ref,
              ws0_ref, bs0_ref, ws1_ref, bs1_ref, ws2_ref, bs2_ref,
              we0_ref, be0_ref, we1_ref, be1_ref, we2_ref, be2_ref,
              sp_ref, ep_ref):
    s = p_ref[0, :NT, :] + p_ref[1, :NT, :] + g_ref[...]
    a = _leaky(dinv_ref[...] * s + b2_ref[...])
    start = _mlp_scores(a, ws0_ref[...], bs0_ref[...], ws1_ref[...],
                        bs1_ref[...], ws2_ref[...], bs2_ref[...])
    end = _mlp_scores(a, we0_ref[...], be0_ref[...], we1_ref[...],
                      be1_ref[...], we2_ref[...], be2_ref[...])
    row = lax.broadcasted_iota(jnp.int32, (NT, 1), 0)
    sp_ref[...] = _softmax0(start, row < N)
    ep_ref[...] = _softmax0(end, jnp.full((NT, 1), True))


_prep = pl.pallas_call(
    _prep_body,
    out_shape=[jax.ShapeDtypeStruct((NT, D), jnp.float32),
               jax.ShapeDtypeStruct((NT, D), jnp.float32)],
)

_mid = pl.pallas_call(
    _mid_body,
    out_shape=jax.ShapeDtypeStruct((NT, D), jnp.float32),
)

_fin = pl.pallas_call(
    _fin_body,
    out_shape=[jax.ShapeDtypeStruct((NT, 1), jnp.float32),
               jax.ShapeDtypeStruct((NT, 1), jnp.float32)],
)


def kernel(x, edge_index, cand, W0, b0, W1, b1, W2, b2,
           Ws0, bs0, Ws1, bs1, Ws2, bs2,
           We0, be0, We1, be1, We2, be2):
    src, dst = edge_index[0], edge_index[1]
    pad = EP - E
    srcp = jnp.concatenate(
        [src, jnp.zeros((pad,), jnp.int32)]).reshape(NW, NCH, K)
    dstp = jnp.concatenate(
        [dst, jnp.full((pad,), TRASH, jnp.int32)]).reshape(NW, NCH, K)
    h = jnp.concatenate([x, cand], axis=0)

    edge_agg, deg_hist = _sc_kernels()
    degp = deg_hist(dstp)
    g0, dinv = _prep(h, W0, degp)
    p0 = edge_agg(g0, srcp, dstp)
    g1 = _mid(p0, g0, dinv, b0, W1)
    p1 = edge_agg(g1, srcp, dstp)
    g2 = _mid(p1, g1, dinv, b1, W2)
    p2 = edge_agg(g2, srcp, dstp)
    sp, ep = _fin(p2, g2, dinv, b2,
                  Ws0, bs0, Ws1, bs1, Ws2, bs2,
                  We0, be0, We1, be1, We2, be2)
    return sp.reshape(NT), ep.reshape(NT)
